# Initial kernel scaffold; baseline (speedup 1.0000x reference)
import jax
import jax.numpy as jnp
from jax.experimental import pallas as pl


def kernel(x, edge_index, W1, b1, W2, b2, W3, b3):
    raise NotImplementedError("write your pallas kernel here")


# speedup vs baseline: 17.6241x; 17.6241x over previous
import jax
import jax.numpy as jnp
from jax import lax
from jax.experimental import pallas as pl
from jax.experimental.pallas import tpu as pltpu
from jax.experimental.pallas import tpu_sc as plsc

N = 10000
E = 320000
D_IN = 128
H1 = 256
H2 = 16
D_OUT = 16

NC = 2
NS = 16
NW = NC * NS
N_PAD = 10240
CHUNK = 128
EPW = 10112
E_PAD = NW * EPW
ROWS_SC = N_PAD // NS
ZROWS = 32

_LANES = 16


def _make_agg(D, gather):
    mesh = plsc.VectorSubcoreMesh(
        core_axis_name="c", subcore_axis_name="s",
        num_cores=NC, num_subcores=NS)
    scratch = [
        pltpu.VMEM((CHUNK,), jnp.int32),
        pltpu.VMEM((CHUNK,), jnp.int32),
        pltpu.VMEM((CHUNK, D), jnp.float32),
        pltpu.VMEM((ZROWS, D), jnp.float32),
        pltpu.VMEM_SHARED((N_PAD, D), jnp.float32),
        pltpu.SemaphoreType.DMA,
    ]

    def body(*refs):
        if gather:
            (h_hbm, src_hbm, dst_hbm, out_hbm,
             src_v, dst_v, rows_v, zb_v, accum, sem) = refs
        else:
            (src_hbm, dst_hbm, out_hbm,
             src_v, dst_v, rows_v, zb_v, accum, sem) = refs
        cid = lax.axis_index("c")
        sid = lax.axis_index("s")
        wid = cid * NS + sid

        zero = jnp.zeros((_LANES,), jnp.float32)
        for i in range(ZROWS):
            for j in range(D // _LANES):
                zb_v[i, pl.ds(j * _LANES, _LANES)] = zero
        if not gather:
            one = jnp.ones((_LANES,), jnp.float32)
            for i in range(CHUNK):
                for j in range(D // _LANES):
                    rows_v[i, pl.ds(j * _LANES, _LANES)] = one

        row0 = sid * ROWS_SC
        for i in range(ROWS_SC // ZROWS):
            pltpu.sync_copy(zb_v, accum.at[pl.ds(row0 + i * ZROWS, ZROWS)])
        plsc.subcore_barrier()

        ebase = wid * EPW

        def step(i, carry):
            b = ebase + i * CHUNK
            if gather:
                pltpu.sync_copy(src_hbm.at[pl.ds(b, CHUNK)], src_v)
            pltpu.sync_copy(dst_hbm.at[pl.ds(b, CHUNK)], dst_v)
            if gather:
                pltpu.async_copy(h_hbm.at[src_v], rows_v, sem).wait()
            pltpu.sync_copy(rows_v, accum.at[dst_v], add=True)
            return carry

        lax.fori_loop(0, EPW // CHUNK, step, 0)
        plsc.subcore_barrier()
        pltpu.sync_copy(accum.at[pl.ds(row0, ROWS_SC)],
                        out_hbm.at[cid, pl.ds(row0, ROWS_SC)])

    return pl.kernel(
        body,
        out_type=jax.ShapeDtypeStruct((NC, N_PAD, D), jnp.float32),
        mesh=mesh,
        scratch_types=scratch,
        compiler_params=pltpu.CompilerParams(use_tc_tiling_on_sc=False),
    )


_agg_deg = _make_agg(16, gather=False)
_agg128 = _make_agg(128, gather=True)
_agg16 = _make_agg(16, gather=True)


_BLK = 512
_GRID = N_PAD // _BLK


def _blk(d):
    return pl.BlockSpec((_BLK, d), lambda i: (i, 0))


def _blk2(d):
    return pl.BlockSpec((NC, _BLK, d), lambda i: (0, i, 0))


def _full(shape):
    nd = len(shape)
    return pl.BlockSpec(shape, lambda i, _n=nd: (0,) * _n)


def _pre_body(dg_ref, x_ref, hp_ref, dinv_ref):
    dinv16 = lax.rsqrt(dg_ref[0] + dg_ref[1] + 1.0)
    dinv_ref[...] = dinv16
    hp_ref[...] = x_ref[...] * dinv16[:, 0:1]


_pre = pl.pallas_call(
    _pre_body,
    grid=(_GRID,),
    in_specs=[_blk2(16), _blk(D_IN)],
    out_specs=[_blk(D_IN), _blk(16)],
    out_shape=[jax.ShapeDtypeStruct((N_PAD, D_IN), jnp.float32),
               jax.ShapeDtypeStruct((N_PAD, 16), jnp.float32)],
)


def _l1_body(a_ref, hp_ref, dinv_ref, w1_ref, b1_ref, w2_ref, out_ref):
    d1 = dinv_ref[...][:, 0:1]
    s = (a_ref[0] + a_ref[1] + hp_ref[...]) * d1
    z1 = jnp.dot(s, w1_ref[...], preferred_element_type=jnp.float32)
    h1 = jnp.maximum(z1 + b1_ref[...], 0.0)
    p = jnp.dot(h1, w2_ref[...], preferred_element_type=jnp.float32)
    out_ref[...] = p * d1


_l1 = pl.pallas_call(
    _l1_body,
    grid=(_GRID,),
    in_specs=[_blk2(D_IN), _blk(D_IN), _blk(16),
              _full((D_IN, H1)), _full((1, H1)), _full((H1, H2))],
    out_specs=_blk(H2),
    out_shape=jax.ShapeDtypeStruct((N_PAD, H2), jnp.float32),
)


def _l2_body(a_ref, pp_ref, dinv_ref, b2_ref, w3_ref, out_ref):
    d1 = dinv_ref[...][:, 0:1]
    h2 = jnp.maximum((a_ref[0] + a_ref[1] + pp_ref[...]) * d1 + b2_ref[...],
                     0.0)
    q = jnp.dot(h2, w3_ref[...], preferred_element_type=jnp.float32)
    out_ref[...] = q * d1


_l2 = pl.pallas_call(
    _l2_body,
    grid=(_GRID,),
    in_specs=[_blk2(H2), _blk(H2), _blk(16),
              _full((1, H2)), _full((H2, D_OUT))],
    out_specs=_blk(D_OUT),
    out_shape=jax.ShapeDtypeStruct((N_PAD, D_OUT), jnp.float32),
)


def _out_body(a_ref, qp_ref, dinv_ref, b3_ref, out_ref):
    d1 = dinv_ref[...][:, 0:1]
    z = (a_ref[0] + a_ref[1] + qp_ref[...]) * d1 + b3_ref[...]
    m = jnp.max(z, axis=1, keepdims=True)
    e = jnp.exp(z - m)
    lse = jnp.log(jnp.sum(e, axis=1, keepdims=True))
    out_ref[...] = (z - m) - lse


_out = pl.pallas_call(
    _out_body,
    grid=(_GRID,),
    in_specs=[_blk2(D_OUT), _blk(D_OUT), _blk(16), _full((1, D_OUT))],
    out_specs=_blk(D_OUT),
    out_shape=jax.ShapeDtypeStruct((N_PAD, D_OUT), jnp.float32),
)


def kernel(x, edge_index, W1, b1, W2, b2, W3, b3):
    src = edge_index[0].astype(jnp.int32)
    dst = edge_index[1].astype(jnp.int32)
    pad_e = E_PAD - E
    pad_idx = (jnp.arange(pad_e, dtype=jnp.int32) % (N_PAD - N)) + N
    src_p = jnp.concatenate([src, pad_idx])
    dst_p = jnp.concatenate([dst, pad_idx])
    x_p = jnp.pad(x, ((0, N_PAD - N), (0, 0)))

    dg = _agg_deg(src_p, dst_p)
    hp, dinv16 = _pre(dg, x_p)
    a1 = _agg128(hp, src_p, dst_p)
    pp = _l1(a1, hp, dinv16, W1, b1.reshape(1, H1), W2)
    a2 = _agg16(pp, src_p, dst_p)
    qp = _l2(a2, pp, dinv16, b2.reshape(1, H2), W3)
    a3 = _agg16(qp, src_p, dst_p)
    outp = _out(a3, qp, dinv16, b3.reshape(1, D_OUT))
    return outp[:N]


# speedup vs baseline: 38.0061x; 2.1565x over previous
import jax
import jax.numpy as jnp
from jax import lax
from jax.experimental import pallas as pl
from jax.experimental.pallas import tpu as pltpu
from jax.experimental.pallas import tpu_sc as plsc

N = 10000
E = 320000
D_IN = 128
H1 = 256
H2 = 16
D_OUT = 16

NC = 2
NS = 16
NW = NC * NS
N_PAD = 10240
EPW = 10240
E_PAD = NW * EPW
ROWS_SC = N_PAD // NS

_LANES = 16


def _make_agg(D, gather, chunk, nbuf):
    nch = EPW // chunk
    ng = nch // nbuf
    assert nch * chunk == EPW and ng * nbuf == nch
    nrows = nbuf if gather else 1
    mesh = plsc.VectorSubcoreMesh(
        core_axis_name="c", subcore_axis_name="s",
        num_cores=NC, num_subcores=NS)
    scratch = [
        *([pltpu.VMEM((nch, chunk), jnp.int32)] if gather else []),
        pltpu.VMEM((nch, chunk), jnp.int32),
        *[pltpu.VMEM((chunk, D), jnp.float32) for _ in range(nrows)],
        pltpu.VMEM_SHARED((N_PAD, D), jnp.float32),
        *[pltpu.SemaphoreType.DMA for _ in range(nbuf)],
    ]

    def body(*refs):
        if gather:
            h_hbm, src_hbm, dst_hbm, out_hbm = refs[:4]
            srcA, dstA = refs[4], refs[5]
            rest = refs[6:]
        else:
            dst_hbm, out_hbm = refs[:2]
            dstA = refs[2]
            rest = refs[3:]
        rows = rest[:nrows]
        accum = rest[nrows]
        sems = rest[nrows + 1:nrows + 1 + nbuf]
        cid = lax.axis_index("c")
        sid = lax.axis_index("s")
        wid = cid * NS + sid

        zero = jnp.zeros((_LANES,), jnp.float32)
        for i in range(chunk):
            for j in range(D // _LANES):
                rows[0][i, pl.ds(j * _LANES, _LANES)] = zero
        row0 = sid * ROWS_SC
        for i in range(ROWS_SC // chunk):
            pltpu.sync_copy(rows[0], accum.at[pl.ds(row0 + i * chunk, chunk)])
        if gather:
            pltpu.sync_copy(src_hbm.at[wid], srcA)
        pltpu.sync_copy(dst_hbm.at[wid], dstA)
        if not gather:
            one = jnp.ones((_LANES,), jnp.float32)
            for i in range(chunk):
                for j in range(D // _LANES):
                    rows[0][i, pl.ds(j * _LANES, _LANES)] = one
        plsc.subcore_barrier()

        if gather:
            for b in range(nbuf):
                pltpu.async_copy(h_hbm.at[srcA.at[b]], rows[b], sems[b])

            def group(g, carry):
                for b in range(nbuf):
                    c = g * nbuf + b
                    pltpu.make_async_copy(
                        h_hbm.at[srcA.at[0]], rows[b], sems[b]).wait()
                    pltpu.sync_copy(rows[b], accum.at[dstA.at[c]], add=True)
                    pltpu.async_copy(
                        h_hbm.at[srcA.at[c + nbuf]], rows[b], sems[b])
                return carry

            lax.fori_loop(0, ng - 1, group, 0)
            for b in range(nbuf):
                c = (ng - 1) * nbuf + b
                pltpu.make_async_copy(
                    h_hbm.at[srcA.at[0]], rows[b], sems[b]).wait()
                pltpu.sync_copy(rows[b], accum.at[dstA.at[c]], add=True)
        else:
            for b in range(nbuf):
                pltpu.async_copy(rows[0], accum.at[dstA.at[b]], sems[b],
                                 add=True)

            def group(g, carry):
                for b in range(nbuf):
                    c = (g + 1) * nbuf + b
                    pltpu.make_async_copy(
                        rows[0], accum.at[dstA.at[0]], sems[b]).wait()
                    pltpu.async_copy(rows[0], accum.at[dstA.at[c]],
                                     sems[b], add=True)
                return carry

            lax.fori_loop(0, ng - 1, group, 0)
            for b in range(nbuf):
                pltpu.make_async_copy(
                    rows[0], accum.at[dstA.at[0]], sems[b]).wait()

        plsc.subcore_barrier()
        pltpu.sync_copy(accum.at[pl.ds(row0, ROWS_SC)],
                        out_hbm.at[cid, pl.ds(row0, ROWS_SC)])

    return pl.kernel(
        body,
        out_type=jax.ShapeDtypeStruct((NC, N_PAD, D), jnp.float32),
        mesh=mesh,
        scratch_types=scratch,
        compiler_params=pltpu.CompilerParams(use_tc_tiling_on_sc=False),
    )


_DEG_CHUNK, _DEG_NBUF = 128, 4
_C128, _B128 = 64, 2
_C16, _B16 = 128, 4
_agg_deg = _make_agg(16, False, _DEG_CHUNK, _DEG_NBUF)
_agg128 = _make_agg(128, True, _C128, _B128)
_agg16 = _make_agg(16, True, _C16, _B16)


_BLK = 512
_GRID = N_PAD // _BLK


def _blk(d):
    return pl.BlockSpec((_BLK, d), lambda i: (i, 0))


def _blk2(d):
    return pl.BlockSpec((NC, _BLK, d), lambda i: (0, i, 0))


def _full(shape):
    nd = len(shape)
    return pl.BlockSpec(shape, lambda i, _n=nd: (0,) * _n)


def _pre_body(dg_ref, x_ref, hp_ref, dinv_ref):
    dinv16 = lax.rsqrt(dg_ref[0] + dg_ref[1] + 1.0)
    dinv_ref[...] = dinv16
    hp_ref[...] = x_ref[...] * dinv16[:, 0:1]


_pre = pl.pallas_call(
    _pre_body,
    grid=(_GRID,),
    in_specs=[_blk2(16), _blk(D_IN)],
    out_specs=[_blk(D_IN), _blk(16)],
    out_shape=[jax.ShapeDtypeStruct((N_PAD, D_IN), jnp.float32),
               jax.ShapeDtypeStruct((N_PAD, 16), jnp.float32)],
)


def _l1_body(a_ref, hp_ref, dinv_ref, w1_ref, b1_ref, w2_ref, out_ref):
    d1 = dinv_ref[...][:, 0:1]
    s = (a_ref[0] + a_ref[1] + hp_ref[...]) * d1
    z1 = jnp.dot(s, w1_ref[...], preferred_element_type=jnp.float32)
    h1 = jnp.maximum(z1 + b1_ref[...], 0.0)
    p = jnp.dot(h1, w2_ref[...], preferred_element_type=jnp.float32)
    out_ref[...] = p * d1


_l1 = pl.pallas_call(
    _l1_body,
    grid=(_GRID,),
    in_specs=[_blk2(D_IN), _blk(D_IN), _blk(16),
              _full((D_IN, H1)), _full((1, H1)), _full((H1, H2))],
    out_specs=_blk(H2),
    out_shape=jax.ShapeDtypeStruct((N_PAD, H2), jnp.float32),
)


def _l2_body(a_ref, pp_ref, dinv_ref, b2_ref, w3_ref, out_ref):
    d1 = dinv_ref[...][:, 0:1]
    h2 = jnp.maximum((a_ref[0] + a_ref[1] + pp_ref[...]) * d1 + b2_ref[...],
                     0.0)
    q = jnp.dot(h2, w3_ref[...], preferred_element_type=jnp.float32)
    out_ref[...] = q * d1


_l2 = pl.pallas_call(
    _l2_body,
    grid=(_GRID,),
    in_specs=[_blk2(H2), _blk(H2), _blk(16),
              _full((1, H2)), _full((H2, D_OUT))],
    out_specs=_blk(D_OUT),
    out_shape=jax.ShapeDtypeStruct((N_PAD, D_OUT), jnp.float32),
)


def _out_body(a_ref, qp_ref, dinv_ref, b3_ref, out_ref):
    d1 = dinv_ref[...][:, 0:1]
    z = (a_ref[0] + a_ref[1] + qp_ref[...]) * d1 + b3_ref[...]
    m = jnp.max(z, axis=1, keepdims=True)
    e = jnp.exp(z - m)
    lse = jnp.log(jnp.sum(e, axis=1, keepdims=True))
    out_ref[...] = (z - m) - lse


_out = pl.pallas_call(
    _out_body,
    grid=(_GRID,),
    in_specs=[_blk2(D_OUT), _blk(D_OUT), _blk(16), _full((1, D_OUT))],
    out_specs=_blk(D_OUT),
    out_shape=jax.ShapeDtypeStruct((N_PAD, D_OUT), jnp.float32),
)


def kernel(x, edge_index, W1, b1, W2, b2, W3, b3):
    src = edge_index[0].astype(jnp.int32)
    dst = edge_index[1].astype(jnp.int32)
    pad_e = E_PAD - E
    pad_idx = (jnp.arange(pad_e, dtype=jnp.int32) % (N_PAD - N)) + N
    src_p = jnp.concatenate([src, pad_idx])
    dst_p = jnp.concatenate([dst, pad_idx])
    s64 = src_p.reshape(NW, EPW // _C128, _C128)
    d64 = dst_p.reshape(NW, EPW // _C128, _C128)
    s128 = src_p.reshape(NW, EPW // _C16, _C16)
    d128 = dst_p.reshape(NW, EPW // _C16, _C16)
    x_p = jnp.pad(x, ((0, N_PAD - N), (0, 0)))

    dg = _agg_deg(d128)
    hp, dinv16 = _pre(dg, x_p)
    a1 = _agg128(hp, s64, d64)
    pp = _l1(a1, hp, dinv16, W1, b1.reshape(1, H1), W2)
    a2 = _agg16(pp, s128, d128)
    qp = _l2(a2, pp, dinv16, b2.reshape(1, H2), W3)
    a3 = _agg16(qp, s128, d128)
    outp = _out(a3, qp, dinv16, b3.reshape(1, D_OUT))
    return outp[:N]


# speedup vs baseline: 41.9904x; 1.1048x over previous
import jax
import jax.numpy as jnp
from jax import lax
from jax.experimental import pallas as pl
from jax.experimental.pallas import tpu as pltpu
from jax.experimental.pallas import tpu_sc as plsc

N = 10000
E = 320000
D_IN = 128
H1 = 256
H2 = 16
D_OUT = 16

NC = 2
NS = 16
NW = NC * NS
N_PAD = 10240
EPW = E // NW
ROWS_SC = N_PAD // NS

_LANES = 16


def _make_agg(D, gather, chunk, nbuf):
    nch = EPW // chunk
    ng = nch // nbuf
    assert nch * chunk == EPW and ng * nbuf == nch
    nrows = nbuf if gather else 1
    mesh = plsc.VectorSubcoreMesh(
        core_axis_name="c", subcore_axis_name="s",
        num_cores=NC, num_subcores=NS)
    scratch = [
        *([pltpu.VMEM((nch, chunk), jnp.int32)] if gather else []),
        pltpu.VMEM((nch, chunk), jnp.int32),
        *[pltpu.VMEM((chunk, D), jnp.float32) for _ in range(nrows)],
        pltpu.VMEM_SHARED((N_PAD, D), jnp.float32),
        *[pltpu.SemaphoreType.DMA for _ in range(nbuf)],
    ]

    def body(*refs):
        if gather:
            h_hbm, src_hbm, dst_hbm, out_hbm = refs[:4]
            srcA, dstA = refs[4], refs[5]
            rest = refs[6:]
        else:
            dst_hbm, out_hbm = refs[:2]
            dstA = refs[2]
            rest = refs[3:]
        rows = rest[:nrows]
        accum = rest[nrows]
        sems = rest[nrows + 1:nrows + 1 + nbuf]
        cid = lax.axis_index("c")
        sid = lax.axis_index("s")
        wid = cid * NS + sid

        zero = jnp.zeros((_LANES,), jnp.float32)
        for i in range(chunk):
            for j in range(D // _LANES):
                rows[0][i, pl.ds(j * _LANES, _LANES)] = zero
        row0 = sid * ROWS_SC
        for i in range(ROWS_SC // chunk):
            pltpu.sync_copy(rows[0], accum.at[pl.ds(row0 + i * chunk, chunk)])
        if gather:
            pltpu.sync_copy(src_hbm.at[wid], srcA)
        pltpu.sync_copy(dst_hbm.at[wid], dstA)
        if not gather:
            one = jnp.ones((_LANES,), jnp.float32)
            for i in range(chunk):
                for j in range(D // _LANES):
                    rows[0][i, pl.ds(j * _LANES, _LANES)] = one
        plsc.subcore_barrier()

        if gather:
            for b in range(nbuf):
                pltpu.async_copy(h_hbm.at[srcA.at[b]], rows[b], sems[b])

            def group(g, carry):
                for b in range(nbuf):
                    c = g * nbuf + b
                    pltpu.make_async_copy(
                        h_hbm.at[srcA.at[0]], rows[b], sems[b]).wait()
                    pltpu.sync_copy(rows[b], accum.at[dstA.at[c]], add=True)
                    pltpu.async_copy(
                        h_hbm.at[srcA.at[c + nbuf]], rows[b], sems[b])
                return carry

            lax.fori_loop(0, ng - 1, group, 0)
            for b in range(nbuf):
                c = (ng - 1) * nbuf + b
                pltpu.make_async_copy(
                    h_hbm.at[srcA.at[0]], rows[b], sems[b]).wait()
                pltpu.sync_copy(rows[b], accum.at[dstA.at[c]], add=True)
        else:
            for b in range(nbuf):
                pltpu.async_copy(rows[0], accum.at[dstA.at[b]], sems[b],
                                 add=True)

            def group(g, carry):
                for b in range(nbuf):
                    c = (g + 1) * nbuf + b
                    pltpu.make_async_copy(
                        rows[0], accum.at[dstA.at[0]], sems[b]).wait()
                    pltpu.async_copy(rows[0], accum.at[dstA.at[c]],
                                     sems[b], add=True)
                return carry

            lax.fori_loop(0, ng - 1, group, 0)
            for b in range(nbuf):
                pltpu.make_async_copy(
                    rows[0], accum.at[dstA.at[0]], sems[b]).wait()

        plsc.subcore_barrier()
        pltpu.sync_copy(accum.at[pl.ds(row0, ROWS_SC)],
                        out_hbm.at[cid, pl.ds(row0, ROWS_SC)])

    return pl.kernel(
        body,
        out_type=jax.ShapeDtypeStruct((NC, N_PAD, D), jnp.float32),
        mesh=mesh,
        scratch_types=scratch,
        compiler_params=pltpu.CompilerParams(use_tc_tiling_on_sc=False),
    )


_CH = 40
_NB = 5
_agg_deg = _make_agg(16, False, _CH, _NB)
_agg128 = _make_agg(128, True, _CH, _NB)
_agg16 = _make_agg(16, True, _CH, _NB)


_BLK = 2048
_GRID = N_PAD // _BLK


def _blk(d):
    return pl.BlockSpec((_BLK, d), lambda i: (i, 0))


def _blk2(d):
    return pl.BlockSpec((NC, _BLK, d), lambda i: (0, i, 0))


def _full(shape):
    nd = len(shape)
    return pl.BlockSpec(shape, lambda i, _n=nd: (0,) * _n)


def _pre_body(dg_ref, x_ref, hp_ref, dinv_ref):
    dinv16 = lax.rsqrt(dg_ref[0] + dg_ref[1] + 1.0)
    dinv_ref[...] = dinv16
    hp_ref[...] = x_ref[...] * dinv16[:, 0:1]


_pre = pl.pallas_call(
    _pre_body,
    grid=(_GRID,),
    in_specs=[_blk2(16), _blk(D_IN)],
    out_specs=[_blk(D_IN), _blk(16)],
    out_shape=[jax.ShapeDtypeStruct((N_PAD, D_IN), jnp.float32),
               jax.ShapeDtypeStruct((N_PAD, 16), jnp.float32)],
)


def _l1_body(a_ref, hp_ref, dinv_ref, w1_ref, b1_ref, w2_ref, out_ref):
    d1 = dinv_ref[...][:, 0:1]
    s = (a_ref[0] + a_ref[1] + hp_ref[...]) * d1
    z1 = jnp.dot(s, w1_ref[...], preferred_element_type=jnp.float32)
    h1 = jnp.maximum(z1 + b1_ref[...], 0.0)
    p = jnp.dot(h1, w2_ref[...], preferred_element_type=jnp.float32)
    out_ref[...] = p * d1


_l1 = pl.pallas_call(
    _l1_body,
    grid=(_GRID,),
    in_specs=[_blk2(D_IN), _blk(D_IN), _blk(16),
              _full((D_IN, H1)), _full((1, H1)), _full((H1, H2))],
    out_specs=_blk(H2),
    out_shape=jax.ShapeDtypeStruct((N_PAD, H2), jnp.float32),
)


def _l2_body(a_ref, pp_ref, dinv_ref, b2_ref, w3_ref, out_ref):
    d1 = dinv_ref[...][:, 0:1]
    h2 = jnp.maximum((a_ref[0] + a_ref[1] + pp_ref[...]) * d1 + b2_ref[...],
                     0.0)
    q = jnp.dot(h2, w3_ref[...], preferred_element_type=jnp.float32)
    out_ref[...] = q * d1


_l2 = pl.pallas_call(
    _l2_body,
    grid=(_GRID,),
    in_specs=[_blk2(H2), _blk(H2), _blk(16),
              _full((1, H2)), _full((H2, D_OUT))],
    out_specs=_blk(D_OUT),
    out_shape=jax.ShapeDtypeStruct((N_PAD, D_OUT), jnp.float32),
)


def _out_body(a_ref, qp_ref, dinv_ref, b3_ref, out_ref):
    d1 = dinv_ref[...][:, 0:1]
    z = (a_ref[0] + a_ref[1] + qp_ref[...]) * d1 + b3_ref[...]
    m = jnp.max(z, axis=1, keepdims=True)
    e = jnp.exp(z - m)
    lse = jnp.log(jnp.sum(e, axis=1, keepdims=True))
    out_ref[...] = (z - m) - lse


_out = pl.pallas_call(
    _out_body,
    grid=(_GRID,),
    in_specs=[_blk2(D_OUT), _blk(D_OUT), _blk(16), _full((1, D_OUT))],
    out_specs=_blk(D_OUT),
    out_shape=jax.ShapeDtypeStruct((N_PAD, D_OUT), jnp.float32),
)


def kernel(x, edge_index, W1, b1, W2, b2, W3, b3):
    sv = edge_index[0].astype(jnp.int32).reshape(NW, EPW // _CH, _CH)
    dv = edge_index[1].astype(jnp.int32).reshape(NW, EPW // _CH, _CH)
    x_p = jnp.pad(x, ((0, N_PAD - N), (0, 0)))

    dg = _agg_deg(dv)
    hp, dinv16 = _pre(dg, x_p)
    a1 = _agg128(hp, sv, dv)
    pp = _l1(a1, hp, dinv16, W1, b1.reshape(1, H1), W2)
    a2 = _agg16(pp, sv, dv)
    qp = _l2(a2, pp, dinv16, b2.reshape(1, H2), W3)
    a3 = _agg16(qp, sv, dv)
    outp = _out(a3, qp, dinv16, b3.reshape(1, D_OUT))
    return outp[:N]
